# baseline (speedup 1.0000x reference)
import jax
import jax.numpy as jnp
from jax.experimental import pallas as pl
from jax.experimental.pallas import tpu as pltpu

_BT = 2048


def _add_kernel(x_ref, p_ref, o_ref):
    o_ref[...] = x_ref[...] + p_ref[...][None, :, :]


def kernel(encoded_tokens, pos_table):
    batch, num_tokens, embed = encoded_tokens.shape
    grid = (num_tokens // _BT, batch)
    return pl.pallas_call(
        _add_kernel,
        grid=grid,
        in_specs=[
            pl.BlockSpec((1, _BT, embed), lambda t, b: (b, t, 0)),
            pl.BlockSpec((_BT, embed), lambda t, b: (t, 0)),
        ],
        out_specs=pl.BlockSpec((1, _BT, embed), lambda t, b: (b, t, 0)),
        out_shape=jax.ShapeDtypeStruct(encoded_tokens.shape, encoded_tokens.dtype),
        compiler_params=pltpu.CompilerParams(
            dimension_semantics=("parallel", "arbitrary"),
        ),
    )(encoded_tokens, pos_table)


# speedup vs baseline: 1.0093x; 1.0093x over previous
import jax
import jax.numpy as jnp
from jax.experimental import pallas as pl
from jax.experimental.pallas import tpu as pltpu

_BT = 1024


def _add_kernel(x_ref, p_ref, o_ref):
    o_ref[...] = x_ref[...] + p_ref[...][None, :, :]


def kernel(encoded_tokens, pos_table):
    batch, num_tokens, embed = encoded_tokens.shape
    grid = (num_tokens // _BT,)
    return pl.pallas_call(
        _add_kernel,
        grid=grid,
        in_specs=[
            pl.BlockSpec((batch, _BT, embed), lambda t: (0, t, 0)),
            pl.BlockSpec((_BT, embed), lambda t: (t, 0)),
        ],
        out_specs=pl.BlockSpec((batch, _BT, embed), lambda t: (0, t, 0)),
        out_shape=jax.ShapeDtypeStruct(encoded_tokens.shape, encoded_tokens.dtype),
        compiler_params=pltpu.CompilerParams(
            dimension_semantics=("arbitrary",),
        ),
    )(encoded_tokens, pos_table)
